# Initial kernel scaffold; baseline (speedup 1.0000x reference)
import jax
import jax.numpy as jnp
from jax.experimental import pallas as pl


def kernel(indices, tables):
    raise NotImplementedError("write your pallas kernel here")


# speedup vs baseline: 9.0490x; 9.0490x over previous
import functools

import jax
import jax.numpy as jnp
from jax import lax
from jax.experimental import pallas as pl
from jax.experimental.pallas import tpu as pltpu
from jax.experimental.pallas import tpu_sc as plsc

B = 16384
F = 26
V = 100000
D = 32
BF = B * F
NW = 32
PER_W = BF // NW
CHUNK = 1024
NCHUNK = PER_W // CHUNK
NSTREAM = CHUNK // 128


def _gather_kernel(table_hbm, idx_hbm, out_hbm, idx_v, fidx_v, rows_v, sem):
    wid = lax.axis_index("s") * 2 + lax.axis_index("c")
    base = wid * PER_W
    lane = lax.iota(jnp.int32, 16)

    def chunk_body(t, carry):
        cb = base + t * CHUNK
        pltpu.sync_copy(idx_hbm.at[pl.ds(cb, CHUNK)], idx_v)
        for j in range(CHUNK // 16):
            pos = cb + (j * 16) + lane
            off = (pos % F) * V
            fidx_v[j // 8, pl.ds((j % 8) * 16, 16)] = idx_v[pl.ds(j * 16, 16)] + off
        copies = [
            pltpu.async_copy(
                table_hbm.at[fidx_v.at[r]],
                rows_v.at[pl.ds(r * 128, 128)],
                sem,
            )
            for r in range(NSTREAM)
        ]
        for c in copies:
            c.wait()
        pltpu.sync_copy(rows_v, out_hbm.at[pl.ds(cb, CHUNK)])
        return carry

    lax.fori_loop(0, NCHUNK, chunk_body, 0)


def kernel(indices, tables):
    flat_tables = tables.reshape(F * V, D)
    flat_idx = indices.reshape(BF)
    mesh = plsc.VectorSubcoreMesh(core_axis_name="c", subcore_axis_name="s")
    run = functools.partial(
        pl.kernel,
        mesh=mesh,
        out_type=jax.ShapeDtypeStruct((BF, D), jnp.float32),
        scratch_types=[
            pltpu.VMEM((CHUNK,), jnp.int32),
            pltpu.VMEM((NSTREAM, 128), jnp.int32),
            pltpu.VMEM((CHUNK, D), jnp.float32),
            pltpu.SemaphoreType.DMA,
        ],
        compiler_params=pltpu.CompilerParams(use_tc_tiling_on_sc=False),
    )(_gather_kernel)
    out = run(flat_tables, flat_idx)
    return out.reshape(B, F * D)
